# baseline (speedup 1.0000x reference)
import functools

import jax
import jax.numpy as jnp
from jax import lax
from jax.experimental import pallas as pl
from jax.experimental.pallas import tpu as pltpu
from jax.experimental.pallas import tpu_sc as plsc

NWORDS_K = 1_000_000
EMB_K = 64
SEQ_K = 200

_NC = 2
_B_PAD = 256
_PER_SCS = _B_PAD // _NC
_LANES = 128
_CW = _B_PAD * _LANES

_V_TILE = 16384
_GRID = (NWORDS_K + _V_TILE - 1) // _V_TILE


def _sc_gather_cols(table_t, idx):
    mesh = plsc.ScalarSubcoreMesh(axis_name="c", num_cores=_NC)

    @functools.partial(
        pl.kernel,
        mesh=mesh,
        out_type=jax.ShapeDtypeStruct((EMB_K, _CW), jnp.float32),
        scratch_types=[
            pltpu.SMEM((_PER_SCS,), jnp.int32),
            pltpu.SemaphoreType.DMA,
        ],
    )
    def k(table_hbm, idx_hbm, out_hbm, idx_s, sem):
        cid = lax.axis_index("c")
        base = cid * _PER_SCS
        pltpu.sync_copy(idx_hbm.at[pl.ds(base, _PER_SCS)], idx_s)
        copies = []
        for j in range(_PER_SCS):
            off = pl.multiple_of(idx_s[j], _LANES)
            for t in range(EMB_K // 8):
                copies.append(pltpu.async_copy(
                    table_hbm.at[pl.ds(8 * t, 8), pl.ds(off, _LANES)],
                    out_hbm.at[pl.ds(8 * t, 8),
                               pl.ds((base + j) * _LANES, _LANES)],
                    sem,
                ))
        for c in copies:
            c.wait()

    return k(table_t, idx)


def _tc_body(u_ref, cols_ref, wt_ref, b_ref, out_ref, s_ref):
    @pl.when(pl.program_id(0) == 0)
    def _():
        s_ref[...] = lax.dot_general(
            u_ref[...], cols_ref[...], (((1,), (1,)), ((), ())),
            preferred_element_type=jnp.float32,
        )

    acc = lax.dot_general(
        s_ref[...], wt_ref[...], (((1,), (0,)), ((), ())),
        preferred_element_type=jnp.float32,
    )
    out_ref[...] = acc + b_ref[...]


def _tc_matvec(u, cols, w_t, b2):
    return pl.pallas_call(
        _tc_body,
        grid=(_GRID,),
        in_specs=[
            pl.BlockSpec((1, _CW), lambda i: (0, 0)),
            pl.BlockSpec((EMB_K, _CW), lambda i: (0, 0)),
            pl.BlockSpec((EMB_K, _V_TILE), lambda i: (0, i)),
            pl.BlockSpec((1, _V_TILE), lambda i: (0, i)),
        ],
        out_specs=pl.BlockSpec((1, _V_TILE), lambda i: (0, i)),
        out_shape=jax.ShapeDtypeStruct((1, NWORDS_K), jnp.float32),
        scratch_shapes=[pltpu.VMEM((1, EMB_K), jnp.float32)],
    )(u, cols, w_t, b2)


def kernel(words, emb_table, proj_W, proj_b):
    w32 = words.astype(jnp.int32)
    idx = jnp.zeros((_B_PAD,), jnp.int32).at[:SEQ_K].set(
        (w32 // _LANES) * _LANES)
    pos = jnp.arange(SEQ_K, dtype=jnp.int32) * _LANES + (w32 % _LANES)
    u = jnp.zeros((_CW,), jnp.float32).at[pos].add(1.0)
    emb_t = emb_table.T
    w_t = proj_W.T
    cols = _sc_gather_cols(emb_t, idx)
    b2 = proj_b.reshape(1, NWORDS_K)
    return _tc_matvec(u.reshape(1, _CW), cols, w_t, b2)


# speedup vs baseline: 3.0424x; 3.0424x over previous
import jax
import jax.numpy as jnp
from jax import lax
from jax.experimental import pallas as pl
from jax.experimental.pallas import tpu as pltpu

NWORDS_K = 1_000_000
EMB_K = 64
SEQ_K = 200

_B_PAD = 256
_LANES = 128
_CW = _B_PAD * _LANES

_V_TILE = 16384
_GRID = (NWORDS_K + _V_TILE - 1) // _V_TILE


def _body(offs_ref, u_ref, emb_any, wt_ref, b_ref, out_ref, cols, s_ref, sem):
    i = pl.program_id(0)

    @pl.when(i == 0)
    def _():
        copies = []
        for j in range(_B_PAD):
            off = pl.multiple_of(offs_ref[j], _LANES)
            copies.append(pltpu.make_async_copy(
                emb_any.at[:, pl.ds(off, _LANES)],
                cols.at[:, pl.ds(j * _LANES, _LANES)],
                sem,
            ))
        for c in copies:
            c.start()
        for c in copies:
            c.wait()
        s_ref[...] = lax.dot_general(
            u_ref[...], cols[...], (((1,), (1,)), ((), ())),
            preferred_element_type=jnp.float32,
        )

    @pl.when(i > 0)
    def _():
        acc = lax.dot_general(
            s_ref[...], wt_ref[...], (((1,), (0,)), ((), ())),
            preferred_element_type=jnp.float32,
        )
        out_ref[...] = acc + b_ref[...]


def kernel(words, emb_table, proj_W, proj_b):
    w32 = words.astype(jnp.int32)
    offs = jnp.zeros((_B_PAD,), jnp.int32).at[:SEQ_K].set(
        (w32 // _LANES) * _LANES)
    pos = jnp.arange(SEQ_K, dtype=jnp.int32) * _LANES + (w32 % _LANES)
    u = jnp.zeros((_CW,), jnp.float32).at[pos].add(1.0).reshape(1, _CW)
    emb_t = emb_table.T
    w_t = proj_W.T
    b2 = proj_b.reshape(1, NWORDS_K)

    grid_spec = pltpu.PrefetchScalarGridSpec(
        num_scalar_prefetch=1,
        grid=(_GRID + 1,),
        in_specs=[
            pl.BlockSpec((1, _CW), lambda i, offs: (0, 0)),
            pl.BlockSpec(memory_space=pl.ANY),
            pl.BlockSpec((EMB_K, _V_TILE),
                         lambda i, offs: (0, jnp.maximum(i - 1, 0))),
            pl.BlockSpec((1, _V_TILE),
                         lambda i, offs: (0, jnp.maximum(i - 1, 0))),
        ],
        out_specs=pl.BlockSpec((1, _V_TILE),
                               lambda i, offs: (0, jnp.maximum(i - 1, 0))),
        scratch_shapes=[
            pltpu.VMEM((EMB_K, _CW), jnp.float32),
            pltpu.VMEM((1, EMB_K), jnp.float32),
            pltpu.SemaphoreType.DMA,
        ],
    )
    return pl.pallas_call(
        _body,
        grid_spec=grid_spec,
        out_shape=jax.ShapeDtypeStruct((1, NWORDS_K), jnp.float32),
    )(offs, u, emb_t, w_t, b2)


# speedup vs baseline: 3.4020x; 1.1182x over previous
import jax
import jax.numpy as jnp
from jax import lax
from jax.experimental import pallas as pl
from jax.experimental.pallas import tpu as pltpu

NWORDS_K = 1_000_000
EMB_K = 64
SEQ_K = 200

_LANES = 128
_CW = SEQ_K * _LANES

_V_TILE = 32768
_GRID = (NWORDS_K + _V_TILE - 1) // _V_TILE


def _body(words_ref, u_ref, emb_any, wt_ref, out_ref, cols, s_ref, sem):
    i = pl.program_id(0)

    @pl.when(i == 0)
    def _():
        copies = []
        for j in range(SEQ_K):
            off = pl.multiple_of((words_ref[j] // _LANES) * _LANES, _LANES)
            copies.append(pltpu.make_async_copy(
                emb_any.at[:, pl.ds(off, _LANES)],
                cols.at[:, pl.ds(j * _LANES, _LANES)],
                sem,
            ))
        for c in copies:
            c.start()
        for c in copies:
            c.wait()
        s_ref[...] = lax.dot_general(
            u_ref[...], cols[...], (((1,), (1,)), ((), ())),
            preferred_element_type=jnp.float32,
        )

    @pl.when(i > 0)
    def _():
        out_ref[...] = lax.dot_general(
            s_ref[...], wt_ref[...], (((1,), (0,)), ((), ())),
            preferred_element_type=jnp.float32,
        )


def kernel(words, emb_table, proj_W, proj_b):
    w32 = words.astype(jnp.int32)
    pos = jnp.arange(SEQ_K, dtype=jnp.int32) * _LANES + (w32 % _LANES)
    u = jnp.zeros((_CW,), jnp.float32).at[pos].add(1.0).reshape(1, _CW)
    emb_t = emb_table.T
    w_t = proj_W.T

    grid_spec = pltpu.PrefetchScalarGridSpec(
        num_scalar_prefetch=1,
        grid=(_GRID + 1,),
        in_specs=[
            pl.BlockSpec((1, _CW), lambda i, w: (0, 0)),
            pl.BlockSpec(memory_space=pl.ANY),
            pl.BlockSpec((EMB_K, _V_TILE),
                         lambda i, w: (0, jnp.maximum(i - 1, 0))),
        ],
        out_specs=pl.BlockSpec((1, _V_TILE),
                               lambda i, w: (0, jnp.maximum(i - 1, 0))),
        scratch_shapes=[
            pltpu.VMEM((EMB_K, _CW), jnp.float32),
            pltpu.VMEM((1, EMB_K), jnp.float32),
            pltpu.SemaphoreType.DMA,
        ],
    )
    out = pl.pallas_call(
        _body,
        grid_spec=grid_spec,
        out_shape=jax.ShapeDtypeStruct((1, NWORDS_K), jnp.float32),
    )(w32, u, emb_t, w_t)
    return out + proj_b[None, :]


# speedup vs baseline: 3.9829x; 1.1708x over previous
import jax
import jax.numpy as jnp
from jax import lax
from jax.experimental import pallas as pl
from jax.experimental.pallas import tpu as pltpu

NWORDS_K = 1_000_000
EMB_K = 64
SEQ_K = 200

_LANES = 128
_CW = SEQ_K * _LANES

_V_TILE = 32768
_GRID = (NWORDS_K + _V_TILE - 1) // _V_TILE


def _body(words_ref, u_ref, emb_any, wt_ref, b_any, out_ref,
          cols, s_ref, b_v, sem, bsem):
    i = pl.program_id(0)

    @pl.when(i == 0)
    def _():
        pltpu.make_async_copy(b_any, b_v, bsem).start()
        copies = []
        for j in range(SEQ_K):
            off = pl.multiple_of((words_ref[j] // _LANES) * _LANES, _LANES)
            copies.append(pltpu.make_async_copy(
                emb_any.at[:, pl.ds(off, _LANES)],
                cols.at[:, pl.ds(j * _LANES, _LANES)],
                sem,
            ))
        for c in copies:
            c.start()
        for c in copies:
            c.wait()
        s_ref[...] = lax.dot_general(
            u_ref[...], cols[...], (((1,), (1,)), ((), ())),
            preferred_element_type=jnp.float32,
        )
        pltpu.make_async_copy(b_any, b_v, bsem).wait()

    @pl.when(i > 0)
    def _():
        acc = lax.dot_general(
            s_ref[...], wt_ref[...], (((1,), (0,)), ((), ())),
            preferred_element_type=jnp.float32,
        )
        b_blk = b_v[pl.ds((i - 1) * _V_TILE, _V_TILE)]
        out_ref[...] = acc + b_blk.reshape(1, _V_TILE)


def kernel(words, emb_table, proj_W, proj_b):
    w32 = words.astype(jnp.int32)
    pos = jnp.arange(SEQ_K, dtype=jnp.int32) * _LANES + (w32 % _LANES)
    u = jnp.zeros((_CW,), jnp.float32).at[pos].add(1.0).reshape(1, _CW)
    emb_t = emb_table.T
    w_t = proj_W.T

    grid_spec = pltpu.PrefetchScalarGridSpec(
        num_scalar_prefetch=1,
        grid=(_GRID + 1,),
        in_specs=[
            pl.BlockSpec((1, _CW), lambda i, w: (0, 0)),
            pl.BlockSpec(memory_space=pl.ANY),
            pl.BlockSpec((EMB_K, _V_TILE),
                         lambda i, w: (0, jnp.maximum(i - 1, 0))),
            pl.BlockSpec(memory_space=pl.ANY),
        ],
        out_specs=pl.BlockSpec((1, _V_TILE),
                               lambda i, w: (0, jnp.maximum(i - 1, 0))),
        scratch_shapes=[
            pltpu.VMEM((EMB_K, _CW), jnp.float32),
            pltpu.VMEM((1, EMB_K), jnp.float32),
            pltpu.VMEM((_GRID * _V_TILE,), jnp.float32),
            pltpu.SemaphoreType.DMA,
            pltpu.SemaphoreType.DMA,
        ],
    )
    return pl.pallas_call(
        _body,
        grid_spec=grid_spec,
        out_shape=jax.ShapeDtypeStruct((1, NWORDS_K), jnp.float32),
    )(w32, u, emb_t, w_t,
      jnp.pad(proj_b, (0, _GRID * _V_TILE - NWORDS_K)))
